# baseline (speedup 1.0000x reference)
import jax
import jax.numpy as jnp
from jax.experimental import pallas as pl
from jax.experimental.pallas import tpu as pltpu


def _smem_spec():
    return pl.BlockSpec(memory_space=pltpu.MemorySpace.SMEM)


def _resident(shape):
    return pl.BlockSpec(shape, lambda i: (0, 0))


def _edge_kernel(eeps_ref, xs_ref, xd_ref, e_ref,
                 w1_ref, b1_ref, w2_ref, b2_ref, w11_ref, b11_ref,
                 e_new_ref, h1a_ref):
    xs = xs_ref[...]
    xd = xd_ref[...]
    e_old = e_ref[...]

    lhs1 = jnp.concatenate([xs, xd, e_old.astype(jnp.bfloat16)], axis=1)
    h = jnp.dot(lhs1, w1_ref[...], preferred_element_type=jnp.float32) + b1_ref[...]
    h = jnp.maximum(h, 0.0)
    e_upd = jnp.dot(h.astype(jnp.bfloat16), w2_ref[...],
                    preferred_element_type=jnp.float32) + b2_ref[...]
    e_new = (1.0 + eeps_ref[0, 0]) * e_old + e_upd
    e_new_ref[...] = e_new

    lhs2 = jnp.concatenate([xs, e_new.astype(jnp.bfloat16)], axis=1)
    h1 = jnp.dot(lhs2, w11_ref[...], preferred_element_type=jnp.float32) + b11_ref[...]
    h1a_ref[:, :256] = jnp.maximum(h1, 0.0)
    h1a_ref[:, 256:] = jnp.ones_like(h1a_ref[:, 256:])


def _make_tc_scatter(n_acc_steps, te, no, f):
    def _kernel(dst_ref, h_ref, acc_out_ref, cnt_out_ref, acc_ref, cnt_ref):
        i = pl.program_id(1)

        @pl.when(i == 0)
        def _init():
            acc_ref[...] = jnp.zeros_like(acc_ref)
            cnt_ref[...] = jnp.zeros_like(cnt_ref)

        @pl.when(i < n_acc_steps)
        def _accumulate():
            iota = jax.lax.broadcasted_iota(jnp.int32, (1, 128), 1)

            def body(k8, _):
                base = pl.multiple_of(k8 * 8, 8)
                chunk = h_ref[pl.ds(base, 8), :]
                for r in range(8):
                    d = dst_ref[0, 0, k8 * 8 + r]
                    row = pltpu.roll(chunk, (8 - r) % 8, axis=0)[0:1, :]
                    acc_ref[d, 0, :] += row[0, :]
                    one = (iota == (d & 127)).astype(jnp.float32)
                    cnt_ref[d >> 7, 0, :] += one[0, :]
                return 0

            jax.lax.fori_loop(0, te // 8, body, 0)

        @pl.when(i >= n_acc_steps)
        def _flush():
            j = i - n_acc_steps
            acc_out_ref[...] = acc_ref[pl.ds(j * no, no), :, :]

        @pl.when(i == n_acc_steps)
        def _flush_cnt():
            cnt_out_ref[...] = cnt_ref[...]

    return _kernel


def _node_kernel(neps_ref, x_ref, s0_ref, s1_ref, aa_ref, ab_ref,
                 ctc_ref, w21x_ref, wc_ref, bc_ref, b21_ref, w22_ref, b22_ref,
                 x_new_ref):
    x_old = x_ref[...]
    s = s0_ref[...] + s1_ref[...]
    a = jnp.concatenate([aa_ref[:, 0, :], ab_ref[:, 0, :]], axis=1)
    sum_h1 = s[:, :256] + a
    cnt = s[:, 256:] + ctc_ref[...]
    mean_h1 = sum_h1 / jnp.maximum(cnt, 1.0)

    h2 = (jnp.dot(x_old.astype(jnp.bfloat16), w21x_ref[...],
                  preferred_element_type=jnp.float32)
          + jnp.dot(mean_h1.astype(jnp.bfloat16), wc_ref[...],
                    preferred_element_type=jnp.float32)
          + jnp.where(cnt > 0.0, bc_ref[...], 0.0)
          + b21_ref[...])
    h2 = jnp.maximum(h2, 0.0)
    x_upd = jnp.dot(h2.astype(jnp.bfloat16), w22_ref[...],
                    preferred_element_type=jnp.float32) + b22_ref[...]
    x_new_ref[...] = (1.0 + neps_ref[0, 0]) * x_old + x_upd


def kernel(x, edge_index, edge_attr, edge_eps, node_eps,
           e_w1_xs, e_w1_xd, e_w1_e, e_b1, e_w2, e_b2,
           n_w11_x, n_w11_e, n_b11, n_w12, n_b12,
           n_w21_x, n_w21_a, n_b21, n_w22, n_b22):
    N, Fx = x.shape
    E, Fe = edge_attr.shape
    H = n_w12.shape[1]
    src, dst = edge_index[0], edge_index[1]

    bf16 = jnp.bfloat16
    xb = x.astype(bf16)

    w1 = jnp.concatenate([e_w1_xs, e_w1_xd, e_w1_e], axis=0).astype(bf16)
    w11 = jnp.concatenate([n_w11_x, n_w11_e], axis=0).astype(bf16)
    w2b = e_w2.astype(bf16)
    wc = jnp.dot(n_w12, n_w21_a).astype(bf16)
    bc = jnp.dot(n_b12, n_w21_a)

    cparams = pltpu.CompilerParams(
        dimension_semantics=("parallel",),
        vmem_limit_bytes=64 * 1024 * 1024,
    )

    TE = 2048

    def run_edge(lo, hi):
        ce = hi - lo
        src_c = src[lo:hi]
        dst_c = dst[lo:hi]
        x_src = jnp.take(xb, src_c, axis=0)
        x_dst = jnp.take(xb, dst_c, axis=0)
        e_new_c, h1a_c = pl.pallas_call(
            _edge_kernel,
            out_shape=(jax.ShapeDtypeStruct((ce, Fe), jnp.float32),
                       jax.ShapeDtypeStruct((ce, H + 1), jnp.float32)),
            grid=(pl.cdiv(ce, TE),),
            in_specs=[
                _smem_spec(),
                pl.BlockSpec((TE, Fx), lambda i: (i, 0)),
                pl.BlockSpec((TE, Fx), lambda i: (i, 0)),
                pl.BlockSpec((TE, Fe), lambda i: (i, 0)),
                _resident(w1.shape), _resident(e_b1.shape),
                _resident(e_w2.shape), _resident(e_b2.shape),
                _resident(w11.shape), _resident(n_b11.shape),
            ],
            out_specs=(pl.BlockSpec((TE, Fe), lambda i: (i, 0)),
                       pl.BlockSpec((TE, H + 1), lambda i: (i, 0))),
            compiler_params=cparams,
        )(edge_eps, x_src, x_dst, edge_attr[lo:hi],
          w1, e_b1, w2b, e_b2, w11, n_b11)
        return e_new_c, h1a_c, dst_c

    E_SC = E // 4
    e_new_0, h1a_0, dst_0 = run_edge(0, E_SC)
    s0 = jax.ops.segment_sum(h1a_0, dst_0, num_segments=N)
    e_new_1, h1a_1, dst_1 = run_edge(E_SC, 2 * E_SC)
    s1 = jax.ops.segment_sum(h1a_1, dst_1, num_segments=N)
    e_new_2, h1a_2, dst_2 = run_edge(2 * E_SC, E)

    E_TC = E - 2 * E_SC
    NB = N // 128
    TS = 2048
    T = E_TC // TS
    F = 16
    NO = N // F
    dst3 = dst_2.reshape(T, 1, TS)
    acc, cntb = pl.pallas_call(
        _make_tc_scatter(T, TS, NO, F),
        out_shape=(jax.ShapeDtypeStruct((2 * N, 1, 128), jnp.float32),
                   jax.ShapeDtypeStruct((2 * NB, 1, 128), jnp.float32)),
        grid=(2, T + F),
        in_specs=[
            pl.BlockSpec((1, 1, TS), lambda c, i: (jnp.minimum(i, T - 1), 0, 0),
                         memory_space=pltpu.MemorySpace.SMEM),
            pl.BlockSpec((TS, 128), lambda c, i: (jnp.minimum(i, T - 1), c)),
        ],
        out_specs=(
            pl.BlockSpec((NO, 1, 128),
                         lambda c, i: (c * F + jnp.clip(i - T, 0, F - 1), 0, 0)),
            pl.BlockSpec((NB, 1, 128), lambda c, i: (c, 0, 0)),
        ),
        scratch_shapes=[
            pltpu.MemorySpace.VMEM((N, 1, 128), jnp.float32),
            pltpu.MemorySpace.VMEM((NB, 1, 128), jnp.float32),
        ],
        compiler_params=pltpu.CompilerParams(
            dimension_semantics=("parallel", "arbitrary"),
            vmem_limit_bytes=48 * 1024 * 1024,
        ),
    )(dst3, h1a_2)
    ctc = cntb[:NB].reshape(N, 1)

    e_new = jnp.concatenate([e_new_0, e_new_1, e_new_2], axis=0)

    TN = 2048
    NBLK = N // TN
    x_new = pl.pallas_call(
        _node_kernel,
        out_shape=jax.ShapeDtypeStruct((N, Fx), jnp.float32),
        grid=(pl.cdiv(N, TN),),
        in_specs=[
            _smem_spec(),
            pl.BlockSpec((TN, Fx), lambda i: (i, 0)),
            pl.BlockSpec((TN, H + 1), lambda i: (i, 0)),
            pl.BlockSpec((TN, H + 1), lambda i: (i, 0)),
            pl.BlockSpec((TN, 1, 128), lambda i: (i, 0, 0)),
            pl.BlockSpec((TN, 1, 128), lambda i: (i + NBLK, 0, 0)),
            pl.BlockSpec((TN, 1), lambda i: (i, 0)),
            _resident((Fx, H)), _resident(wc.shape), _resident(bc.shape),
            _resident(n_b21.shape), _resident((H, Fx)), _resident(n_b22.shape),
        ],
        out_specs=pl.BlockSpec((TN, Fx), lambda i: (i, 0)),
        compiler_params=cparams,
    )(node_eps, x, s0, s1, acc, acc, ctc,
      n_w21_x.astype(bf16), wc, bc, n_b21, n_w22.astype(bf16), n_b22)

    return x_new, e_new


# speedup vs baseline: 1.1243x; 1.1243x over previous
import jax
import jax.numpy as jnp
from jax.experimental import pallas as pl
from jax.experimental.pallas import tpu as pltpu


def _smem_spec():
    return pl.BlockSpec(memory_space=pltpu.MemorySpace.SMEM)


def _resident(shape):
    return pl.BlockSpec(shape, lambda i: (0, 0))


def _edge_kernel(eeps_ref, xs_ref, xd_ref, e_ref,
                 w1_ref, b1_ref, w2_ref, b2_ref, w11_ref, b11_ref,
                 e_new_ref, h1a_ref):
    xs = xs_ref[...]
    xd = xd_ref[...]
    e_old = e_ref[...]

    lhs1 = jnp.concatenate([xs, xd, e_old.astype(jnp.bfloat16)], axis=1)
    h = jnp.dot(lhs1, w1_ref[...], preferred_element_type=jnp.float32) + b1_ref[...]
    h = jnp.maximum(h, 0.0)
    e_upd = jnp.dot(h.astype(jnp.bfloat16), w2_ref[...],
                    preferred_element_type=jnp.float32) + b2_ref[...]
    e_new = (1.0 + eeps_ref[0, 0]) * e_old + e_upd
    e_new_ref[...] = e_new

    lhs2 = jnp.concatenate([xs, e_new.astype(jnp.bfloat16)], axis=1)
    h1 = jnp.dot(lhs2, w11_ref[...], preferred_element_type=jnp.float32) + b11_ref[...]
    h1a_ref[:, :256] = jnp.maximum(h1, 0.0)
    h1a_ref[:, 256:] = jnp.ones_like(h1a_ref[:, 256:])


def _make_tc_scatter(n_acc_steps, te, no, f):
    def _kernel(dst_ref, h_ref, acc_out_ref, cnt_out_ref, acc_ref, cnt_ref):
        i = pl.program_id(1)

        @pl.when(i == 0)
        def _init():
            acc_ref[...] = jnp.zeros_like(acc_ref)
            cnt_ref[...] = jnp.zeros_like(cnt_ref)

        @pl.when(i < n_acc_steps)
        def _accumulate():
            iota = jax.lax.broadcasted_iota(jnp.int32, (1, 128), 1)

            def body(k8, _):
                base = pl.multiple_of(k8 * 8, 8)
                chunk = h_ref[pl.ds(base, 8), :]
                for r in range(8):
                    d = dst_ref[0, 0, k8 * 8 + r]
                    row = pltpu.roll(chunk, (8 - r) % 8, axis=0)[0:1, :]
                    acc_ref[d, 0, :] += row[0, :]
                    one = (iota == (d & 127)).astype(jnp.float32)
                    cnt_ref[d >> 7, 0, :] += one[0, :]
                return 0

            jax.lax.fori_loop(0, te // 8, body, 0)

        @pl.when(i >= n_acc_steps)
        def _flush():
            j = i - n_acc_steps
            acc_out_ref[...] = acc_ref[pl.ds(j * no, no), :, :]

        @pl.when(i == n_acc_steps)
        def _flush_cnt():
            cnt_out_ref[...] = cnt_ref[...]

    return _kernel


def _node_kernel(neps_ref, x_ref, s0_ref, s1_ref, s2_ref, aa_ref, ab_ref,
                 ctc_ref, w21x_ref, wc_ref, bc_ref, b21_ref, w22_ref, b22_ref,
                 x_new_ref):
    x_old = x_ref[...]
    s = s0_ref[...] + s1_ref[...] + s2_ref[...]
    a = jnp.concatenate([aa_ref[:, 0, :], ab_ref[:, 0, :]], axis=1)
    sum_h1 = s[:, :256] + a
    cnt = s[:, 256:] + ctc_ref[...]
    mean_h1 = sum_h1 / jnp.maximum(cnt, 1.0)

    h2 = (jnp.dot(x_old.astype(jnp.bfloat16), w21x_ref[...],
                  preferred_element_type=jnp.float32)
          + jnp.dot(mean_h1.astype(jnp.bfloat16), wc_ref[...],
                    preferred_element_type=jnp.float32)
          + jnp.where(cnt > 0.0, bc_ref[...], 0.0)
          + b21_ref[...])
    h2 = jnp.maximum(h2, 0.0)
    x_upd = jnp.dot(h2.astype(jnp.bfloat16), w22_ref[...],
                    preferred_element_type=jnp.float32) + b22_ref[...]
    x_new_ref[...] = (1.0 + neps_ref[0, 0]) * x_old + x_upd


def kernel(x, edge_index, edge_attr, edge_eps, node_eps,
           e_w1_xs, e_w1_xd, e_w1_e, e_b1, e_w2, e_b2,
           n_w11_x, n_w11_e, n_b11, n_w12, n_b12,
           n_w21_x, n_w21_a, n_b21, n_w22, n_b22):
    N, Fx = x.shape
    E, Fe = edge_attr.shape
    H = n_w12.shape[1]
    src, dst = edge_index[0], edge_index[1]

    bf16 = jnp.bfloat16
    xb = x.astype(bf16)

    w1 = jnp.concatenate([e_w1_xs, e_w1_xd, e_w1_e], axis=0).astype(bf16)
    w11 = jnp.concatenate([n_w11_x, n_w11_e], axis=0).astype(bf16)
    w2b = e_w2.astype(bf16)
    wc = jnp.dot(n_w12, n_w21_a).astype(bf16)
    bc = jnp.dot(n_b12, n_w21_a)

    cparams = pltpu.CompilerParams(
        dimension_semantics=("parallel",),
        vmem_limit_bytes=64 * 1024 * 1024,
    )

    TE = 2048

    def run_edge(lo, hi):
        ce = hi - lo
        src_c = src[lo:hi]
        dst_c = dst[lo:hi]
        x_src = jnp.take(xb, src_c, axis=0)
        x_dst = jnp.take(xb, dst_c, axis=0)
        e_new_c, h1a_c = pl.pallas_call(
            _edge_kernel,
            out_shape=(jax.ShapeDtypeStruct((ce, Fe), jnp.float32),
                       jax.ShapeDtypeStruct((ce, H + 1), jnp.float32)),
            grid=(pl.cdiv(ce, TE),),
            in_specs=[
                _smem_spec(),
                pl.BlockSpec((TE, Fx), lambda i: (i, 0)),
                pl.BlockSpec((TE, Fx), lambda i: (i, 0)),
                pl.BlockSpec((TE, Fe), lambda i: (i, 0)),
                _resident(w1.shape), _resident(e_b1.shape),
                _resident(e_w2.shape), _resident(e_b2.shape),
                _resident(w11.shape), _resident(n_b11.shape),
            ],
            out_specs=(pl.BlockSpec((TE, Fe), lambda i: (i, 0)),
                       pl.BlockSpec((TE, H + 1), lambda i: (i, 0))),
            compiler_params=cparams,
        )(edge_eps, x_src, x_dst, edge_attr[lo:hi],
          w1, e_b1, w2b, e_b2, w11, n_b11)
        return e_new_c, h1a_c, dst_c

    E_SC = E // 4
    e_new_0, h1a_0, dst_0 = run_edge(0, E_SC)
    s0 = jax.ops.segment_sum(h1a_0, dst_0, num_segments=N)
    e_new_1, h1a_1, dst_1 = run_edge(E_SC, 2 * E_SC)
    s1 = jax.ops.segment_sum(h1a_1, dst_1, num_segments=N)
    e_new_1b, h1a_1b, dst_1b = run_edge(2 * E_SC, 3 * E_SC)
    s1b = jax.ops.segment_sum(h1a_1b, dst_1b, num_segments=N)
    e_new_2, h1a_2, dst_2 = run_edge(3 * E_SC, E)

    E_TC = E - 3 * E_SC
    NB = N // 128
    TS = 2048
    T = E_TC // TS
    F = 16
    NO = N // F
    dst3 = dst_2.reshape(T, 1, TS)
    acc, cntb = pl.pallas_call(
        _make_tc_scatter(T, TS, NO, F),
        out_shape=(jax.ShapeDtypeStruct((2 * N, 1, 128), jnp.float32),
                   jax.ShapeDtypeStruct((2 * NB, 1, 128), jnp.float32)),
        grid=(2, T + F),
        in_specs=[
            pl.BlockSpec((1, 1, TS), lambda c, i: (jnp.minimum(i, T - 1), 0, 0),
                         memory_space=pltpu.MemorySpace.SMEM),
            pl.BlockSpec((TS, 128), lambda c, i: (jnp.minimum(i, T - 1), c)),
        ],
        out_specs=(
            pl.BlockSpec((NO, 1, 128),
                         lambda c, i: (c * F + jnp.clip(i - T, 0, F - 1), 0, 0)),
            pl.BlockSpec((NB, 1, 128), lambda c, i: (c, 0, 0)),
        ),
        scratch_shapes=[
            pltpu.MemorySpace.VMEM((N, 1, 128), jnp.float32),
            pltpu.MemorySpace.VMEM((NB, 1, 128), jnp.float32),
        ],
        compiler_params=pltpu.CompilerParams(
            dimension_semantics=("parallel", "arbitrary"),
            vmem_limit_bytes=48 * 1024 * 1024,
        ),
    )(dst3, h1a_2)
    ctc = cntb[:NB].reshape(N, 1)

    e_new = jnp.concatenate([e_new_0, e_new_1, e_new_1b, e_new_2], axis=0)

    TN = 2048
    NBLK = N // TN
    x_new = pl.pallas_call(
        _node_kernel,
        out_shape=jax.ShapeDtypeStruct((N, Fx), jnp.float32),
        grid=(pl.cdiv(N, TN),),
        in_specs=[
            _smem_spec(),
            pl.BlockSpec((TN, Fx), lambda i: (i, 0)),
            pl.BlockSpec((TN, H + 1), lambda i: (i, 0)),
            pl.BlockSpec((TN, H + 1), lambda i: (i, 0)),
            pl.BlockSpec((TN, H + 1), lambda i: (i, 0)),
            pl.BlockSpec((TN, 1, 128), lambda i: (i, 0, 0)),
            pl.BlockSpec((TN, 1, 128), lambda i: (i + NBLK, 0, 0)),
            pl.BlockSpec((TN, 1), lambda i: (i, 0)),
            _resident((Fx, H)), _resident(wc.shape), _resident(bc.shape),
            _resident(n_b21.shape), _resident((H, Fx)), _resident(n_b22.shape),
        ],
        out_specs=pl.BlockSpec((TN, Fx), lambda i: (i, 0)),
        compiler_params=cparams,
    )(node_eps, x, s0, s1, s1b, acc, acc, ctc,
      n_w21_x.astype(bf16), wc, bc, n_b21, n_w22.astype(bf16), n_b22)

    return x_new, e_new


# speedup vs baseline: 1.1351x; 1.0096x over previous
import jax
import jax.numpy as jnp
from jax.experimental import pallas as pl
from jax.experimental.pallas import tpu as pltpu


def _smem_spec():
    return pl.BlockSpec(memory_space=pltpu.MemorySpace.SMEM)


def _resident(shape):
    return pl.BlockSpec(shape, lambda i: (0, 0))


def _edge_kernel(eeps_ref, xs_ref, xd_ref, e_ref,
                 w1_ref, b1_ref, w2_ref, b2_ref, w11_ref, b11_ref,
                 e_new_ref, h1a_ref):
    xs = xs_ref[...]
    xd = xd_ref[...]
    e_old = e_ref[...]

    lhs1 = jnp.concatenate([xs, xd, e_old.astype(jnp.bfloat16)], axis=1)
    h = jnp.dot(lhs1, w1_ref[...], preferred_element_type=jnp.float32) + b1_ref[...]
    h = jnp.maximum(h, 0.0)
    e_upd = jnp.dot(h.astype(jnp.bfloat16), w2_ref[...],
                    preferred_element_type=jnp.float32) + b2_ref[...]
    e_new = (1.0 + eeps_ref[0, 0]) * e_old + e_upd
    e_new_ref[...] = e_new

    lhs2 = jnp.concatenate([xs, e_new.astype(jnp.bfloat16)], axis=1)
    h1 = jnp.dot(lhs2, w11_ref[...], preferred_element_type=jnp.float32) + b11_ref[...]
    h1a_ref[:, :256] = jnp.maximum(h1, 0.0)
    h1a_ref[:, 256:] = jnp.ones_like(h1a_ref[:, 256:])


def _make_tc_scatter(n_acc_steps, te, no, f):
    def _kernel(dst_ref, h_ref, acc_out_ref, cnt_out_ref, acc_ref, cnt_ref):
        i = pl.program_id(1)

        @pl.when(i == 0)
        def _init():
            acc_ref[...] = jnp.zeros_like(acc_ref)
            cnt_ref[...] = jnp.zeros_like(cnt_ref)

        @pl.when(i < n_acc_steps)
        def _accumulate():
            iota = jax.lax.broadcasted_iota(jnp.int32, (1, 128), 1)

            def body(k8, _):
                base = pl.multiple_of(k8 * 8, 8)
                chunk = h_ref[pl.ds(base, 8), :]
                for r in range(8):
                    d = dst_ref[0, 0, k8 * 8 + r]
                    row = pltpu.roll(chunk, (8 - r) % 8, axis=0)[0:1, :]
                    acc_ref[d, 0, :] += row[0, :]
                    one = (iota == (d & 127)).astype(jnp.float32)
                    cnt_ref[d >> 7, 0, :] += one[0, :]
                return 0

            jax.lax.fori_loop(0, te // 8, body, 0)

        @pl.when(i >= n_acc_steps)
        def _flush():
            j = i - n_acc_steps
            acc_out_ref[...] = acc_ref[pl.ds(j * no, no), :, :]

        @pl.when(i == n_acc_steps)
        def _flush_cnt():
            cnt_out_ref[...] = cnt_ref[...]

    return _kernel


def _node_kernel(neps_ref, x_ref, s0_ref, s1_ref, aa_ref, ab_ref,
                 ctc_ref, w21x_ref, wc_ref, bc_ref, b21_ref, w22_ref, b22_ref,
                 x_new_ref):
    x_old = x_ref[...]
    s = s0_ref[...] + s1_ref[...]
    a = jnp.concatenate([aa_ref[:, 0, :], ab_ref[:, 0, :]], axis=1)
    sum_h1 = s[:, :256] + a
    cnt = s[:, 256:] + ctc_ref[...]
    mean_h1 = sum_h1 / jnp.maximum(cnt, 1.0)

    h2 = (jnp.dot(x_old.astype(jnp.bfloat16), w21x_ref[...],
                  preferred_element_type=jnp.float32)
          + jnp.dot(mean_h1.astype(jnp.bfloat16), wc_ref[...],
                    preferred_element_type=jnp.float32)
          + jnp.where(cnt > 0.0, bc_ref[...], 0.0)
          + b21_ref[...])
    h2 = jnp.maximum(h2, 0.0)
    x_upd = jnp.dot(h2.astype(jnp.bfloat16), w22_ref[...],
                    preferred_element_type=jnp.float32) + b22_ref[...]
    x_new_ref[...] = (1.0 + neps_ref[0, 0]) * x_old + x_upd


def kernel(x, edge_index, edge_attr, edge_eps, node_eps,
           e_w1_xs, e_w1_xd, e_w1_e, e_b1, e_w2, e_b2,
           n_w11_x, n_w11_e, n_b11, n_w12, n_b12,
           n_w21_x, n_w21_a, n_b21, n_w22, n_b22):
    N, Fx = x.shape
    E, Fe = edge_attr.shape
    H = n_w12.shape[1]
    src, dst = edge_index[0], edge_index[1]

    bf16 = jnp.bfloat16
    xb = x.astype(bf16)

    w1 = jnp.concatenate([e_w1_xs, e_w1_xd, e_w1_e], axis=0).astype(bf16)
    w11 = jnp.concatenate([n_w11_x, n_w11_e], axis=0).astype(bf16)
    w2b = e_w2.astype(bf16)
    wc = jnp.dot(n_w12, n_w21_a).astype(bf16)
    bc = jnp.dot(n_b12, n_w21_a)

    cparams = pltpu.CompilerParams(
        dimension_semantics=("parallel",),
        vmem_limit_bytes=64 * 1024 * 1024,
    )

    TE = 2048

    def run_edge(lo, hi):
        ce = hi - lo
        src_c = src[lo:hi]
        dst_c = dst[lo:hi]
        x_src = jnp.take(xb, src_c, axis=0)
        x_dst = jnp.take(xb, dst_c, axis=0)
        e_new_c, h1a_c = pl.pallas_call(
            _edge_kernel,
            out_shape=(jax.ShapeDtypeStruct((ce, Fe), jnp.float32),
                       jax.ShapeDtypeStruct((ce, H + 1), jnp.float32)),
            grid=(pl.cdiv(ce, TE),),
            in_specs=[
                _smem_spec(),
                pl.BlockSpec((TE, Fx), lambda i: (i, 0)),
                pl.BlockSpec((TE, Fx), lambda i: (i, 0)),
                pl.BlockSpec((TE, Fe), lambda i: (i, 0)),
                _resident(w1.shape), _resident(e_b1.shape),
                _resident(e_w2.shape), _resident(e_b2.shape),
                _resident(w11.shape), _resident(n_b11.shape),
            ],
            out_specs=(pl.BlockSpec((TE, Fe), lambda i: (i, 0)),
                       pl.BlockSpec((TE, H + 1), lambda i: (i, 0))),
            compiler_params=cparams,
        )(edge_eps, x_src, x_dst, edge_attr[lo:hi],
          w1, e_b1, w2b, e_b2, w11, n_b11)
        return e_new_c, h1a_c, dst_c

    C1 = 3 * E // 8
    C2 = 3 * E // 4
    e_new_0, h1a_0, dst_0 = run_edge(0, C1)
    s0 = jax.ops.segment_sum(h1a_0, dst_0, num_segments=N)
    e_new_1, h1a_1, dst_1 = run_edge(C1, C2)
    s1 = jax.ops.segment_sum(h1a_1, dst_1, num_segments=N)
    e_new_2, h1a_2, dst_2 = run_edge(C2, E)

    E_TC = E - C2
    NB = N // 128
    TS = 2048
    T = E_TC // TS
    F = 16
    NO = N // F
    dst3 = dst_2.reshape(T, 1, TS)
    acc, cntb = pl.pallas_call(
        _make_tc_scatter(T, TS, NO, F),
        out_shape=(jax.ShapeDtypeStruct((2 * N, 1, 128), jnp.float32),
                   jax.ShapeDtypeStruct((2 * NB, 1, 128), jnp.float32)),
        grid=(2, T + F),
        in_specs=[
            pl.BlockSpec((1, 1, TS), lambda c, i: (jnp.minimum(i, T - 1), 0, 0),
                         memory_space=pltpu.MemorySpace.SMEM),
            pl.BlockSpec((TS, 128), lambda c, i: (jnp.minimum(i, T - 1), c)),
        ],
        out_specs=(
            pl.BlockSpec((NO, 1, 128),
                         lambda c, i: (c * F + jnp.clip(i - T, 0, F - 1), 0, 0)),
            pl.BlockSpec((NB, 1, 128), lambda c, i: (c, 0, 0)),
        ),
        scratch_shapes=[
            pltpu.MemorySpace.VMEM((N, 1, 128), jnp.float32),
            pltpu.MemorySpace.VMEM((NB, 1, 128), jnp.float32),
        ],
        compiler_params=pltpu.CompilerParams(
            dimension_semantics=("parallel", "arbitrary"),
            vmem_limit_bytes=48 * 1024 * 1024,
        ),
    )(dst3, h1a_2)
    ctc = cntb[:NB].reshape(N, 1)

    e_new = jnp.concatenate([e_new_0, e_new_1, e_new_2], axis=0)

    TN = 2048
    NBLK = N // TN
    x_new = pl.pallas_call(
        _node_kernel,
        out_shape=jax.ShapeDtypeStruct((N, Fx), jnp.float32),
        grid=(pl.cdiv(N, TN),),
        in_specs=[
            _smem_spec(),
            pl.BlockSpec((TN, Fx), lambda i: (i, 0)),
            pl.BlockSpec((TN, H + 1), lambda i: (i, 0)),
            pl.BlockSpec((TN, H + 1), lambda i: (i, 0)),
            pl.BlockSpec((TN, 1, 128), lambda i: (i, 0, 0)),
            pl.BlockSpec((TN, 1, 128), lambda i: (i + NBLK, 0, 0)),
            pl.BlockSpec((TN, 1), lambda i: (i, 0)),
            _resident((Fx, H)), _resident(wc.shape), _resident(bc.shape),
            _resident(n_b21.shape), _resident((H, Fx)), _resident(n_b22.shape),
        ],
        out_specs=pl.BlockSpec((TN, Fx), lambda i: (i, 0)),
        compiler_params=cparams,
    )(node_eps, x, s0, s1, acc, acc, ctc,
      n_w21_x.astype(bf16), wc, bc, n_b21, n_w22.astype(bf16), n_b22)

    return x_new, e_new
